# Initial kernel scaffold; baseline (speedup 1.0000x reference)
import jax
import jax.numpy as jnp
from jax.experimental import pallas as pl


def kernel(ob_no, histogram, n):
    raise NotImplementedError("write your pallas kernel here")


# speedup vs baseline: 1.6041x; 1.6041x over previous
import functools

import jax
import jax.numpy as jnp
from jax import lax
from jax.experimental import pallas as pl
from jax.experimental.pallas import tpu as pltpu
from jax.experimental.pallas import tpu_sc as plsc

H = 1024
W = 1024
M = H * W
N = 1048576
NC, NS, L = 2, 16, 16
NW = NC * NS
T = N // NW
P = 4096
PIECES = T // P
SC_CHUNK = M // NS


def _sc_body(ob_hbm, hist_hbm, scale_hbm, bonus_hbm, a_hbm, b_hbm,
             obp, idxb, cnt, ones, sclv, shared, sem):
    c = lax.axis_index("c")
    s = lax.axis_index("s")
    wid = s * NC + c
    lane = lax.iota(jnp.int32, 16)

    def _fill(i, carry):
        ones[pl.ds(i * 16, 16)] = jnp.full((16,), 1.0, jnp.float32)
        cnt[pl.ds(i * 16, 16)] = jnp.zeros((16,), jnp.float32)
        return carry
    lax.fori_loop(0, P // 16, _fill, 0)

    for k in range(SC_CHUNK // P):
        pltpu.sync_copy(cnt, shared.at[pl.ds(s * SC_CHUNK + k * P, P)])
    pltpu.sync_copy(scale_hbm, sclv)
    plsc.subcore_barrier()

    sv = sclv[...]
    for p in range(PIECES):
        off = wid * T + p * P
        pltpu.sync_copy(ob_hbm.at[pl.ds(off * 2, 2 * P)], obp)

        swp = lane ^ 1
        ev2 = (lane & 7) * 2
        mco = jnp.where((lane & 1) == 0, W, 1).astype(jnp.int32)

        def _mkidx(i, carry):
            b2 = i * 32
            v1 = obp[pl.ds(b2, 16)]
            v2 = obp[pl.ds(b2 + 16, 16)]
            u1 = jnp.minimum(jnp.maximum(v1.astype(jnp.int32), 0), H - 1)
            u2 = jnp.minimum(jnp.maximum(v2.astype(jnp.int32), 0), H - 1)
            w1 = u1 * mco
            w2 = u2 * mco
            s1 = w1 + w1.at[swp].get(mode="promise_in_bounds")
            s2 = w2 + w2.at[swp].get(mode="promise_in_bounds")
            e1 = s1.at[ev2].get(mode="promise_in_bounds")
            e2 = s2.at[ev2].get(mode="promise_in_bounds")
            idxb[pl.ds(i * 16, 16)] = jnp.where(lane < 8, e1, e2)
            return carry
        lax.fori_loop(0, P // 16, _mkidx, 0)

        gat = pltpu.async_copy(hist_hbm.at[idxb], cnt, sem)
        pltpu.sync_copy(ones, shared.at[idxb], add=True)
        gat.wait()

        def _bonus(i, carry):
            x = cnt[pl.ds(i * 16, 16)] + 1.0
            ib = lax.bitcast_convert_type(x, jnp.int32)
            ib = 0x5F3759DF - lax.shift_right_logical(ib, 1)
            y = lax.bitcast_convert_type(ib, jnp.float32)
            y = y * (1.5 - 0.5 * x * y * y)
            y = y * (1.5 - 0.5 * x * y * y)
            y = y * (1.5 - 0.5 * x * y * y)
            cnt[pl.ds(i * 16, 16)] = y * sv
            return carry
        lax.fori_loop(0, P // 16, _bonus, 0)
        pltpu.sync_copy(cnt, bonus_hbm.at[pl.ds(off, P)])

    plsc.subcore_barrier()

    @pl.when(c == 0)
    def _():
        pltpu.sync_copy(shared.at[pl.ds(s * SC_CHUNK, SC_CHUNK)],
                        a_hbm.at[pl.ds(s * SC_CHUNK, SC_CHUNK)])

    @pl.when(c == 1)
    def _():
        pltpu.sync_copy(shared.at[pl.ds(s * SC_CHUNK, SC_CHUNK)],
                        b_hbm.at[pl.ds(s * SC_CHUNK, SC_CHUNK)])


_sc_call = functools.partial(
    pl.kernel,
    out_type=(
        jax.ShapeDtypeStruct((N,), jnp.float32),
        jax.ShapeDtypeStruct((M,), jnp.float32),
        jax.ShapeDtypeStruct((M,), jnp.float32),
    ),
    mesh=plsc.VectorSubcoreMesh(core_axis_name="c", subcore_axis_name="s"),
    scratch_types=[
        pltpu.VMEM((2 * P,), jnp.float32),
        pltpu.VMEM((P,), jnp.int32),
        pltpu.VMEM((P,), jnp.float32),
        pltpu.VMEM((P,), jnp.float32),
        pltpu.VMEM((16,), jnp.float32),
        pltpu.VMEM_SHARED((M,), jnp.float32),
        pltpu.SemaphoreType.DMA,
    ],
)(_sc_body)


def _combine_body(h_ref, a_ref, b_ref, o_ref):
    o_ref[...] = h_ref[...] + (a_ref[...] + b_ref[...])


_combine = pl.pallas_call(
    _combine_body,
    grid=(8,),
    in_specs=[pl.BlockSpec((H // 8, W), lambda i: (i, 0))] * 3,
    out_specs=pl.BlockSpec((H // 8, W), lambda i: (i, 0)),
    out_shape=jax.ShapeDtypeStruct((H, W), jnp.float32),
)


def kernel(ob_no, histogram, n):
    new_n = jnp.float32(n + N)
    scale = jnp.sqrt(2.0 * jnp.log(new_n))
    scale_arr = jnp.full((16,), scale, jnp.float32)
    bonus, part_a, part_b = _sc_call(
        ob_no.reshape(-1), histogram.reshape(-1), scale_arr)
    new_hist = _combine(histogram, part_a.reshape(H, W), part_b.reshape(H, W))
    return bonus, new_hist


# speedup vs baseline: 21.4911x; 13.3977x over previous
import functools

import jax
import jax.numpy as jnp
from jax import lax
from jax.experimental import pallas as pl
from jax.experimental.pallas import tpu as pltpu
from jax.experimental.pallas import tpu_sc as plsc

H = 1024
W = 1024
M = H * W
N = 1048576
NC, NS = 2, 16
NW = NC * NS
T = N // NW
P = 8192
PIECES = T // P
SC_CHUNK = M // NS
GB = 8


def _sc_body(ob_hbm, hist_hbm, cnt_hbm, a_hbm, b_hbm,
             obr, obc, idxb, cnt, ones, shared, sem):
    c = lax.axis_index("c")
    s = lax.axis_index("s")
    wid = s * NC + c

    def _fill(i, carry):
        ones[pl.ds(i * 16, 16)] = jnp.full((16,), 1.0, jnp.float32)
        cnt[pl.ds(i * 16, 16)] = jnp.zeros((16,), jnp.float32)
        return carry
    lax.fori_loop(0, P // 16, _fill, 0)

    for k in range(SC_CHUNK // P):
        pltpu.sync_copy(cnt, shared.at[pl.ds(s * SC_CHUNK + k * P, P)])
    plsc.subcore_barrier()

    for p in range(PIECES):
        off = wid * T + p * P
        pltpu.sync_copy(ob_hbm.at[pl.ds(off, P)], obr)
        pltpu.sync_copy(ob_hbm.at[pl.ds(N + off, P)], obc)

        def _mkidx(i, carry):
            r = obr[pl.ds(i * 16, 16)]
            q = obc[pl.ds(i * 16, 16)]
            ri = jnp.minimum(jnp.maximum(r.astype(jnp.int32), 0), H - 1)
            ci = jnp.minimum(jnp.maximum(q.astype(jnp.int32), 0), W - 1)
            idxb[pl.ds(i * 16, 16)] = ri * W + ci
            return carry
        lax.fori_loop(0, P // 16, _mkidx, 0)

        gat = pltpu.async_copy(hist_hbm.at[idxb], cnt, sem)
        pltpu.sync_copy(ones, shared.at[idxb], add=True)
        gat.wait()
        pltpu.sync_copy(cnt, cnt_hbm.at[pl.ds(off, P)])

    plsc.subcore_barrier()

    @pl.when(c == 0)
    def _():
        pltpu.sync_copy(shared.at[pl.ds(s * SC_CHUNK, SC_CHUNK)],
                        a_hbm.at[pl.ds(s * SC_CHUNK, SC_CHUNK)])

    @pl.when(c == 1)
    def _():
        pltpu.sync_copy(shared.at[pl.ds(s * SC_CHUNK, SC_CHUNK)],
                        b_hbm.at[pl.ds(s * SC_CHUNK, SC_CHUNK)])


_sc_call = functools.partial(
    pl.kernel,
    out_type=(
        jax.ShapeDtypeStruct((N,), jnp.float32),
        jax.ShapeDtypeStruct((M,), jnp.float32),
        jax.ShapeDtypeStruct((M,), jnp.float32),
    ),
    mesh=plsc.VectorSubcoreMesh(core_axis_name="c", subcore_axis_name="s"),
    scratch_types=[
        pltpu.VMEM((P,), jnp.float32),
        pltpu.VMEM((P,), jnp.float32),
        pltpu.VMEM((P,), jnp.int32),
        pltpu.VMEM((P,), jnp.float32),
        pltpu.VMEM((P,), jnp.float32),
        pltpu.VMEM_SHARED((M,), jnp.float32),
        pltpu.SemaphoreType.DMA,
    ],
)(_sc_body)


def _combine_body(s2_ref, h_ref, a_ref, b_ref, cnt_ref, oh_ref, ob_ref):
    part = (a_ref[...] + b_ref[...]).reshape(H // GB, W)
    oh_ref[...] = h_ref[...] + part
    ob_ref[...] = jnp.sqrt(s2_ref[0] / (cnt_ref[...] + 1.0))


_combine = pl.pallas_call(
    _combine_body,
    grid=(GB,),
    in_specs=[
        pl.BlockSpec(memory_space=pltpu.MemorySpace.SMEM),
        pl.BlockSpec((H // GB, W), lambda i: (i, 0)),
        pl.BlockSpec((M // GB,), lambda i: (i,)),
        pl.BlockSpec((M // GB,), lambda i: (i,)),
        pl.BlockSpec((N // GB,), lambda i: (i,)),
    ],
    out_specs=[
        pl.BlockSpec((H // GB, W), lambda i: (i, 0)),
        pl.BlockSpec((N // GB,), lambda i: (i,)),
    ],
    out_shape=(
        jax.ShapeDtypeStruct((H, W), jnp.float32),
        jax.ShapeDtypeStruct((N,), jnp.float32),
    ),
)


def kernel(ob_no, histogram, n):
    s2 = jnp.reshape(2.0 * jnp.log(jnp.float32(n + N)), (1,))
    cnt, part_a, part_b = _sc_call(
        ob_no.T.reshape(-1), histogram.reshape(-1))
    new_hist, bonus = _combine(s2, histogram, part_a, part_b, cnt)
    return bonus, new_hist


# speedup vs baseline: 22.3880x; 1.0417x over previous
import functools

import jax
import jax.numpy as jnp
from jax import lax
from jax.experimental import pallas as pl
from jax.experimental.pallas import tpu as pltpu
from jax.experimental.pallas import tpu_sc as plsc

H = 1024
W = 1024
M = H * W
N = 1048576
NC, NS = 2, 16
NW = NC * NS
T = N // NW
P = 4096
PIECES = T // P
SC_CHUNK = M // NS
GB = 8


def _sc_body(ob_hbm, hist_hbm, cnt_hbm, a_hbm, b_hbm,
             obr, obc, idxb, cnt, ones, shared, sob, sga, sout):
    c = lax.axis_index("c")
    s = lax.axis_index("s")
    wid = s * NC + c

    def _fill(i, carry):
        ones[pl.ds(i * 16, 16)] = jnp.full((16,), 1.0, jnp.float32)
        cnt[0][pl.ds(i * 16, 16)] = jnp.zeros((16,), jnp.float32)
        return carry
    lax.fori_loop(0, P // 16, _fill, 0)

    for k in range(SC_CHUNK // P):
        pltpu.sync_copy(cnt[0], shared.at[pl.ds(s * SC_CHUNK + k * P, P)])
    plsc.subcore_barrier()

    def _start_obs(p, b):
        off = wid * T + p * P
        return (pltpu.async_copy(ob_hbm.at[pl.ds(off, P)], obr[b], sob),
                pltpu.async_copy(ob_hbm.at[pl.ds(N + off, P)], obc[b], sob))

    obs_dma = _start_obs(0, 0)
    out_dma = [None, None]
    for p in range(PIECES):
        b = p % 2
        off = wid * T + p * P
        obs_dma[0].wait()
        obs_dma[1].wait()
        if p + 1 < PIECES:
            obs_dma = _start_obs(p + 1, 1 - b)

        def _mkidx(i, carry):
            r = obr[b][pl.ds(i * 16, 16)]
            q = obc[b][pl.ds(i * 16, 16)]
            ri = jnp.minimum(jnp.maximum(r.astype(jnp.int32), 0), H - 1)
            ci = jnp.minimum(jnp.maximum(q.astype(jnp.int32), 0), W - 1)
            idxb[pl.ds(i * 16, 16)] = ri * W + ci
            return carry
        lax.fori_loop(0, P // 16, _mkidx, 0)

        if out_dma[b] is not None:
            out_dma[b].wait()
        gat = pltpu.async_copy(hist_hbm.at[idxb], cnt[b], sga)
        pltpu.sync_copy(ones, shared.at[idxb], add=True)
        gat.wait()
        out_dma[b] = pltpu.async_copy(cnt[b], cnt_hbm.at[pl.ds(off, P)], sout)

    for d in out_dma:
        if d is not None:
            d.wait()
    plsc.subcore_barrier()

    @pl.when(c == 0)
    def _():
        pltpu.sync_copy(shared.at[pl.ds(s * SC_CHUNK, SC_CHUNK)],
                        a_hbm.at[pl.ds(s * SC_CHUNK, SC_CHUNK)])

    @pl.when(c == 1)
    def _():
        pltpu.sync_copy(shared.at[pl.ds(s * SC_CHUNK, SC_CHUNK)],
                        b_hbm.at[pl.ds(s * SC_CHUNK, SC_CHUNK)])


_sc_call = functools.partial(
    pl.kernel,
    out_type=(
        jax.ShapeDtypeStruct((N,), jnp.float32),
        jax.ShapeDtypeStruct((M,), jnp.float32),
        jax.ShapeDtypeStruct((M,), jnp.float32),
    ),
    mesh=plsc.VectorSubcoreMesh(core_axis_name="c", subcore_axis_name="s"),
    scratch_types=[
        [pltpu.VMEM((P,), jnp.float32)] * 2,
        [pltpu.VMEM((P,), jnp.float32)] * 2,
        pltpu.VMEM((P,), jnp.int32),
        [pltpu.VMEM((P,), jnp.float32)] * 2,
        pltpu.VMEM((P,), jnp.float32),
        pltpu.VMEM_SHARED((M,), jnp.float32),
        pltpu.SemaphoreType.DMA,
        pltpu.SemaphoreType.DMA,
        pltpu.SemaphoreType.DMA,
    ],
)(_sc_body)


def _combine_body(s2_ref, h_ref, a_ref, b_ref, cnt_ref, oh_ref, ob_ref):
    part = (a_ref[...] + b_ref[...]).reshape(H // GB, W)
    oh_ref[...] = h_ref[...] + part
    ob_ref[...] = jnp.sqrt(s2_ref[0] / (cnt_ref[...] + 1.0))


_combine = pl.pallas_call(
    _combine_body,
    grid=(GB,),
    in_specs=[
        pl.BlockSpec(memory_space=pltpu.MemorySpace.SMEM),
        pl.BlockSpec((H // GB, W), lambda i: (i, 0)),
        pl.BlockSpec((M // GB,), lambda i: (i,)),
        pl.BlockSpec((M // GB,), lambda i: (i,)),
        pl.BlockSpec((N // GB,), lambda i: (i,)),
    ],
    out_specs=[
        pl.BlockSpec((H // GB, W), lambda i: (i, 0)),
        pl.BlockSpec((N // GB,), lambda i: (i,)),
    ],
    out_shape=(
        jax.ShapeDtypeStruct((H, W), jnp.float32),
        jax.ShapeDtypeStruct((N,), jnp.float32),
    ),
)


def kernel(ob_no, histogram, n):
    s2 = jnp.reshape(2.0 * jnp.log(jnp.float32(n + N)), (1,))
    cnt, part_a, part_b = _sc_call(
        ob_no.T.reshape(-1), histogram.reshape(-1))
    new_hist, bonus = _combine(s2, histogram, part_a, part_b, cnt)
    return bonus, new_hist
